# baseline (speedup 1.0000x reference)
import functools

import jax
import jax.numpy as jnp
from jax import lax
from jax.experimental import pallas as pl
from jax.experimental.pallas import tpu as pltpu
from jax.experimental.pallas import tpu_sc as plsc

N = 10000
DF = 128
H = 16
NCLS = 10
E = 320000
NC, NS, L = 2, 16, 16
NW = NC * NS
EPT = E // NW
CHUNK = 128
FCPT = EPT // CHUNK
TAIL = EPT - FCPT * CHUNK
TAIL_OFF = FCPT * CHUNK
NPAD = 10240
R128 = NPAD // 128
STRIPE = NPAD // NS
SROWS = STRIPE * H // 128

_mesh = plsc.VectorSubcoreMesh(
    core_axis_name="c", subcore_axis_name="s", num_cores=NC, num_subcores=NS)
_sc_params = pltpu.CompilerParams(
    needs_layout_passes=False, use_tc_tiling_on_sc=False)


@functools.partial(
    pl.kernel,
    out_type=jax.ShapeDtypeStruct((NC, R128, 128), jnp.float32),
    mesh=_mesh,
    compiler_params=_sc_params,
    scratch_types=[
        pltpu.VMEM((EPT,), jnp.int32),
        pltpu.VMEM((R128, 128), jnp.float32),
        pltpu.VMEM((R128 // NS, 128), jnp.float32),
        pltpu.VMEM((R128,), jnp.int32),
        pltpu.VMEM_SHARED((R128, 128), jnp.float32),
    ],
)
def _deg_kernel(dst_hbm, deg_out, dstv, degv, z5, iota80, deg_sp):
    cid = lax.axis_index("c")
    sid = lax.axis_index("s")
    wid = sid * NC + cid
    pltpu.sync_copy(dst_hbm.at[pl.ds(wid * EPT, EPT)], dstv)

    zero = jnp.full((L,), 0.0, jnp.float32)
    one = jnp.full((L,), 1.0, jnp.float32)

    def _zero(i, _):
        degv[i // 8, pl.ds((i % 8) * L, L)] = zero
        return 0

    lax.fori_loop(0, NPAD // L, _zero, 0)
    for g in range((R128 // NS) * 8):
        z5[g // 8, pl.ds((g % 8) * L, L)] = zero
    for k in range(R128 // L):
        iota80[pl.ds(k * L, L)] = lax.iota(jnp.int32, L) + k * L
    pltpu.sync_copy(z5, deg_sp.at[pl.ds(sid * (R128 // NS), R128 // NS)])

    @plsc.parallel_loop(0, EPT // L, unroll=4)
    def _acc(i):
        idx = dstv[pl.ds(i * L, L)]
        plsc.addupdate_scatter(degv, [idx >> 7, idx & 127], one)

    plsc.subcore_barrier()
    pltpu.sync_copy(degv, deg_sp.at[iota80], add=True)
    plsc.subcore_barrier()

    @pl.when(sid == 0)
    def _():
        pltpu.sync_copy(deg_sp, deg_out.at[cid])


def _mm_body(x_ref, w1_ref, xw_ref):
    xw = jnp.dot(x_ref[...], w1_ref[...],
                 preferred_element_type=jnp.float32)
    xw_ref[...] = jnp.concatenate(
        [xw, jnp.zeros((NPAD - N, H), jnp.float32)], axis=0)


def _mm_kernel(x, w1):
    return pl.pallas_call(
        _mm_body,
        out_shape=jax.ShapeDtypeStruct((NPAD, H), jnp.float32),
    )(x, w1)


@functools.partial(
    pl.kernel,
    out_type=(
        jax.ShapeDtypeStruct((NC, NS * SROWS, 128), jnp.float32),
        jax.ShapeDtypeStruct((NC, NS * SROWS, 128), jnp.float32),
    ),
    mesh=_mesh,
    compiler_params=_sc_params,
    scratch_types=[
        pltpu.VMEM((EPT,), jnp.int32),
        pltpu.VMEM((EPT,), jnp.int32),
        pltpu.VMEM((4, CHUNK), jnp.int32),
        pltpu.VMEM((TAIL,), jnp.int32),
        pltpu.VMEM((R128, 128), jnp.float32),
        pltpu.VMEM((R128, 128), jnp.float32),
        pltpu.VMEM((NC, R128 // NS, 128), jnp.float32),
        pltpu.VMEM((STRIPE, H), jnp.float32),
        pltpu.VMEM((STRIPE, H), jnp.float32),
        pltpu.VMEM((R128 // NS, 128), jnp.float32),
        pltpu.VMEM((4, CHUNK, H), jnp.float32),
        pltpu.VMEM((TAIL, H), jnp.float32),
        pltpu.VMEM((SROWS, 128), jnp.float32),
        pltpu.VMEM((SROWS, 128), jnp.float32),
        pltpu.VMEM((R128,), jnp.int32),
        pltpu.VMEM_SHARED((NPAD, H), jnp.float32),
        pltpu.VMEM_SHARED((NPAD, H), jnp.float32),
        pltpu.VMEM_SHARED((R128, 128), jnp.float32),
        pltpu.VMEM_SHARED((R128, 128), jnp.float32),
        pltpu.SemaphoreType.DMA,
        pltpu.SemaphoreType.DMA,
        pltpu.SemaphoreType.DMA,
        pltpu.SemaphoreType.DMA,
        pltpu.SemaphoreType.DMA,
        pltpu.SemaphoreType.DMA,
        pltpu.SemaphoreType.DMA,
        pltpu.SemaphoreType.DMA,
        pltpu.SemaphoreType.DMA,
        pltpu.SemaphoreType.DMA,
        pltpu.SemaphoreType.DMA,
        pltpu.SemaphoreType.DMA,
    ],
)
def _edge_kernel(src_hbm, dst_hbm, deg_hbm, xw_hbm,
                 h1p_out, cexp_out,
                 srcf, dstf, didx, tidx, disv, caccv, degstg, ybuf, h1stg,
                 s128buf, rows, tailrows, hout, cout, iota80, h1sh, y_sp,
                 dis_sp, cacc_sp,
                 gs0, gs1, gs2, gs3, ss0, ss1, ss2, ss3,
                 ds0, ds1, ds2, ds3):
    gs = (gs0, gs1, gs2, gs3)
    ss = (ss0, ss1, ss2, ss3)
    dsm = (ds0, ds1, ds2, ds3)
    cid = lax.axis_index("c")
    sid = lax.axis_index("s")
    wid = sid * NC + cid
    ebase = wid * EPT
    nbase = sid * STRIPE

    pltpu.sync_copy(src_hbm.at[pl.ds(ebase, EPT)], srcf)
    pltpu.sync_copy(dst_hbm.at[pl.ds(ebase, EPT)], dstf)

    degcps = [
        pltpu.async_copy(
            deg_hbm.at[k, pl.ds(sid * (R128 // NS), R128 // NS)],
            degstg.at[k], gs0)
        for k in range(NC)
    ]
    zero = jnp.full((L,), 0.0, jnp.float32)

    def _zero_cacc(i, _):
        caccv[i // 8, pl.ds((i % 8) * L, L)] = zero
        return 0

    lax.fori_loop(0, NPAD // L, _zero_cacc, 0)

    def _zero_rows(i, _):
        rows[0, i, :] = zero
        return 0

    lax.fori_loop(0, CHUNK, _zero_rows, 0)
    for k in range(STRIPE // CHUNK):
        pltpu.sync_copy(rows.at[0], h1sh.at[pl.ds(nbase + k * CHUNK, CHUNK)])
    for d in degcps:
        d.wait()

    def _dis_stripe(i, _):
        q, o = i // 8, (i % 8) * L
        deg = (degstg[0, q, pl.ds(o, L)] + degstg[1, q, pl.ds(o, L)] + 1.0)
        yh = plsc.bitcast(
            jnp.full((L,), 0x5F3759DF, jnp.int32)
            - (plsc.bitcast(deg, jnp.int32) >> 1),
            jnp.float32)
        half = deg * -0.5
        for _ in range(3):
            yh = yh * (half * yh * yh + 1.5)
        s128buf[i // 8, pl.ds((i % 8) * L, L)] = yh
        return 0

    lax.fori_loop(0, STRIPE // L, _dis_stripe, 0)
    pltpu.sync_copy(s128buf, dis_sp.at[pl.ds(sid * (R128 // NS), R128 // NS)])
    plsc.subcore_barrier()

    pltpu.sync_copy(dis_sp, disv)
    pltpu.sync_copy(xw_hbm.at[pl.ds(nbase, STRIPE)], ybuf)

    def _ymul(g, _):
        nloc = nbase + g * L
        dv = disv[nloc // 128, pl.ds(nloc % 128, L)]
        for t in range(L):
            ybuf[g * L + t, :] = ybuf[g * L + t, :] * dv[t]
        return 0

    lax.fori_loop(0, STRIPE // L, _ymul, 0)
    pltpu.sync_copy(ybuf, y_sp.at[pl.ds(nbase, STRIPE)])
    plsc.subcore_barrier()

    NBUF, LA = 4, 2

    def _start_gather(c, b):
        return pltpu.async_copy(
            y_sp.at[srcf.at[pl.ds(c * CHUNK, CHUNK)]], rows.at[b], gs[b])

    def _start_didx(c, b):
        return pltpu.async_copy(
            dst_hbm.at[pl.ds(ebase + c * CHUNK, CHUNK)], didx.at[b], dsm[b])

    def _start_scatter(c, b):
        return pltpu.async_copy(
            rows.at[b], h1sh.at[didx.at[b]], ss[b], add=True)

    def _wait_gather(b):
        pltpu.make_async_copy(
            y_sp.at[srcf.at[pl.ds(0, CHUNK)]], rows.at[b], gs[b]).wait()

    def _wait_didx(b):
        pltpu.make_async_copy(
            dst_hbm.at[pl.ds(0, CHUNK)], didx.at[b], dsm[b]).wait()

    def _wait_scatter(b):
        pltpu.make_async_copy(rows.at[b], h1sh.at[didx.at[b]], ss[b]).wait()

    def _cacc_chunk(c):
        @plsc.parallel_loop(0, CHUNK // L, unroll=4)
        def _cacc(g):
            e0 = c * CHUNK + g * L
            di = dstf[pl.ds(e0, L)]
            si = srcf[pl.ds(e0, L)]
            vals = plsc.load_gather(disv, [di >> 7, di & 127])
            plsc.addupdate_scatter(caccv, [si >> 7, si & 127], vals)

    MAIN = (FCPT - LA) // NBUF * NBUF
    for j in range(LA):
        _start_didx(j, j)
        _start_gather(j, j)

    def _main(i, _):
        c0 = i * NBUF
        for j in range(NBUF):
            c = c0 + j
            _wait_gather(j)
            _wait_didx(j)
            _cacc_chunk(c)
            _start_scatter(c, j)
            nb = (j + LA) % NBUF

            @pl.when(c >= LA)
            def _():
                _wait_scatter(nb)

            _start_didx(c + LA, nb)
            _start_gather(c + LA, nb)
        return 0

    lax.fori_loop(0, MAIN // NBUF, _main, 0)

    for c in range(MAIN, FCPT):
        b = c % NBUF
        _wait_gather(b)
        _wait_didx(b)
        _cacc_chunk(c)
        _start_scatter(c, b)
        nc = c + LA
        if nc < FCPT:
            nb = nc % NBUF
            _wait_scatter(nb)
            _start_didx(nc, nb)
            _start_gather(nc, nb)
    pltpu.sync_copy(dst_hbm.at[pl.ds(ebase + TAIL_OFF, TAIL)], tidx)
    pltpu.async_copy(
        y_sp.at[srcf.at[pl.ds(TAIL_OFF, TAIL)]], tailrows, gs0).wait()
    pltpu.async_copy(tailrows, h1sh.at[tidx], ss0, add=True).wait()

    @plsc.parallel_loop(0, TAIL // L, unroll=1)
    def _cacc_tail(g):
        e0 = TAIL_OFF + g * L
        di = dstf[pl.ds(e0, L)]
        si = srcf[pl.ds(e0, L)]
        vals = plsc.load_gather(disv, [di >> 7, di & 127])
        plsc.addupdate_scatter(caccv, [si >> 7, si & 127], vals)

    for b in range(NBUF):
        _wait_scatter(b)
    plsc.subcore_barrier()

    @pl.when(wid == 0)
    def _():
        def _sladd(i, _):
            q, o = i // 8, (i % 8) * L
            caccv[q, pl.ds(o, L)] = (caccv[q, pl.ds(o, L)]
                                     + disv[q, pl.ds(o, L)])
            return 0

        lax.fori_loop(0, N // L, _sladd, 0)

    def _cmul(i, _):
        q, o = i // 8, (i % 8) * L
        caccv[q, pl.ds(o, L)] = caccv[q, pl.ds(o, L)] * disv[q, pl.ds(o, L)]
        return 0

    lax.fori_loop(0, NPAD // L, _cmul, 0)
    for k in range(R128 // L):
        iota80[pl.ds(k * L, L)] = lax.iota(jnp.int32, L) + k * L

    @pl.when(sid == 0)
    def _():
        def _zc(i, _):
            s128buf[i // 8, pl.ds((i % 8) * L, L)] = zero
            return 0

        lax.fori_loop(0, STRIPE // L, _zc, 0)
        for k in range(NS):
            pltpu.sync_copy(s128buf,
                            cacc_sp.at[pl.ds(k * (R128 // NS), R128 // NS)])

    plsc.subcore_barrier()
    pltpu.sync_copy(caccv, cacc_sp.at[iota80], add=True)
    plsc.subcore_barrier()

    pltpu.sync_copy(h1sh.at[pl.ds(nbase, STRIPE)], h1stg)
    pltpu.sync_copy(cacc_sp.at[pl.ds(sid * (R128 // NS), R128 // NS)], s128buf)
    yfac = jnp.where(cid == 0, 1.0, 0.0).astype(jnp.float32)

    def _emit(g, _):
        j0 = g * L
        nloc = nbase + j0
        dv = disv[nloc // 128, pl.ds(nloc % 128, L)]
        cv = s128buf[j0 // 128, pl.ds(j0 % 128, L)]
        for t in range(L):
            j = j0 + t
            q = 2 * g + t // 8
            i = t % 8
            h1row = (h1stg[j, :] + ybuf[j, :] * yfac) * dv[t]
            hout[q, pl.ds(i * L, L)] = h1row
            cout[q, pl.ds(i * L, L)] = jnp.zeros((L,), jnp.float32) + cv[t]
        return 0

    lax.fori_loop(0, STRIPE // L, _emit, 0)
    pltpu.sync_copy(hout, h1p_out.at[cid, pl.ds(sid * SROWS, SROWS)])
    pltpu.sync_copy(cout, cexp_out.at[cid, pl.ds(sid * SROWS, SROWS)])


def _fin_body(h1p_ref, cexp_ref, b1_ref, w2_ref, b2_ref, out_ref):
    b1t = jnp.concatenate([b1_ref[...]] * (128 // H), axis=1)
    h1 = jnp.maximum(h1p_ref[0] + h1p_ref[1] + b1t, 0.0)
    ce = cexp_ref[0] + cexp_ref[1]
    s128 = jnp.sum(ce * h1, axis=0, keepdims=True)
    row = lax.broadcasted_iota(jnp.int32, (128, H), 0)
    col = lax.broadcasted_iota(jnp.int32, (128, H), 1)
    p128 = jnp.where(row % H == col, 1.0, 0.0).astype(jnp.float32)
    s16 = jnp.dot(s128, p128,
                  preferred_element_type=jnp.float32)
    logits = jnp.dot(s16 * (1.0 / N), w2_ref[...],
                     preferred_element_type=jnp.float32) + b2_ref[...]
    m = jnp.max(logits)
    ex = jnp.exp(logits - m)
    out_ref[...] = ex / jnp.sum(ex)


def _fin_kernel(h1p, cexp, b1, w2, b2):
    return pl.pallas_call(
        _fin_body,
        out_shape=jax.ShapeDtypeStruct((1, NCLS), jnp.float32),
    )(h1p, cexp, b1, w2, b2)


def kernel(x, edge_index, W1, b1, W2, b2):
    ei = edge_index.astype(jnp.int32)
    src = ei[0]
    dst = ei[1]
    deg = _deg_kernel(dst)
    xw = _mm_kernel(x, W1)
    h1p, cexp = _edge_kernel(src, dst, deg, xw)
    out = _fin_kernel(h1p, cexp, b1.reshape(1, H), W2, b2.reshape(1, NCLS))
    return out.reshape(NCLS)


# speedup vs baseline: 1.1323x; 1.1323x over previous
import functools

import jax
import jax.numpy as jnp
from jax import lax
from jax.experimental import pallas as pl
from jax.experimental.pallas import tpu as pltpu
from jax.experimental.pallas import tpu_sc as plsc

N = 10000
DF = 128
H = 16
NCLS = 10
E = 320000
NC, NS, L = 2, 16, 16
NW = NC * NS
EPT = E // NW
CHUNK = 128
FCPT = EPT // CHUNK
TAIL = EPT - FCPT * CHUNK
TAIL_OFF = FCPT * CHUNK
NPAD = 10240
R128 = NPAD // 128
STRIPE = NPAD // NS
SROWS = STRIPE * H // 128

_mesh = plsc.VectorSubcoreMesh(
    core_axis_name="c", subcore_axis_name="s", num_cores=NC, num_subcores=NS)
_sc_params = pltpu.CompilerParams(
    needs_layout_passes=False, use_tc_tiling_on_sc=False)


@functools.partial(
    pl.kernel,
    out_type=jax.ShapeDtypeStruct((NC, R128, 128), jnp.float32),
    mesh=_mesh,
    compiler_params=_sc_params,
    scratch_types=[
        pltpu.VMEM((EPT,), jnp.int32),
        pltpu.VMEM((R128, 128), jnp.float32),
        pltpu.VMEM((R128 // NS, 128), jnp.float32),
        pltpu.VMEM((R128,), jnp.int32),
        pltpu.VMEM_SHARED((R128, 128), jnp.float32),
    ],
)
def _deg_kernel(ei_hbm, deg_out, dstv, degv, z5, iota80, deg_sp):
    cid = lax.axis_index("c")
    sid = lax.axis_index("s")
    wid = sid * NC + cid
    pltpu.sync_copy(ei_hbm.at[1, pl.ds(wid * EPT, EPT)], dstv)

    zero = jnp.full((L,), 0.0, jnp.float32)
    one = jnp.full((L,), 1.0, jnp.float32)

    def _zero(i, _):
        degv[i // 8, pl.ds((i % 8) * L, L)] = zero
        return 0

    lax.fori_loop(0, NPAD // L, _zero, 0)
    for g in range((R128 // NS) * 8):
        z5[g // 8, pl.ds((g % 8) * L, L)] = zero
    for k in range(R128 // L):
        iota80[pl.ds(k * L, L)] = lax.iota(jnp.int32, L) + k * L
    pltpu.sync_copy(z5, deg_sp.at[pl.ds(sid * (R128 // NS), R128 // NS)])

    @plsc.parallel_loop(0, EPT // L, unroll=4)
    def _acc(i):
        idx = dstv[pl.ds(i * L, L)]
        plsc.addupdate_scatter(degv, [idx >> 7, idx & 127], one)

    plsc.subcore_barrier()
    pltpu.sync_copy(degv, deg_sp.at[iota80], add=True)
    plsc.subcore_barrier()

    @pl.when(sid == 0)
    def _():
        pltpu.sync_copy(deg_sp, deg_out.at[cid])


def _mm_body(x_ref, w1_ref, xw_ref):
    xw = jnp.dot(x_ref[...], w1_ref[...],
                 preferred_element_type=jnp.float32)
    xw_ref[...] = jnp.concatenate(
        [xw, jnp.zeros((NPAD - N, H), jnp.float32)], axis=0)


def _mm_kernel(x, w1):
    return pl.pallas_call(
        _mm_body,
        out_shape=jax.ShapeDtypeStruct((NPAD, H), jnp.float32),
    )(x, w1)


@functools.partial(
    pl.kernel,
    out_type=(
        jax.ShapeDtypeStruct((NC, NS * SROWS, 128), jnp.float32),
        jax.ShapeDtypeStruct((NC, NS * SROWS, 128), jnp.float32),
    ),
    mesh=_mesh,
    compiler_params=_sc_params,
    scratch_types=[
        pltpu.VMEM((EPT,), jnp.int32),
        pltpu.VMEM((EPT,), jnp.int32),
        pltpu.VMEM((4, CHUNK), jnp.int32),
        pltpu.VMEM((TAIL,), jnp.int32),
        pltpu.VMEM((R128, 128), jnp.float32),
        pltpu.VMEM((R128, 128), jnp.float32),
        pltpu.VMEM((NC, R128 // NS, 128), jnp.float32),
        pltpu.VMEM((STRIPE, H), jnp.float32),
        pltpu.VMEM((STRIPE, H), jnp.float32),
        pltpu.VMEM((R128 // NS, 128), jnp.float32),
        pltpu.VMEM((4, CHUNK, H), jnp.float32),
        pltpu.VMEM((TAIL, H), jnp.float32),
        pltpu.VMEM((SROWS, 128), jnp.float32),
        pltpu.VMEM((SROWS, 128), jnp.float32),
        pltpu.VMEM((R128,), jnp.int32),
        pltpu.VMEM_SHARED((NPAD, H), jnp.float32),
        pltpu.VMEM_SHARED((NPAD, H), jnp.float32),
        pltpu.VMEM_SHARED((R128, 128), jnp.float32),
        pltpu.VMEM_SHARED((R128, 128), jnp.float32),
        pltpu.SemaphoreType.DMA,
        pltpu.SemaphoreType.DMA,
        pltpu.SemaphoreType.DMA,
        pltpu.SemaphoreType.DMA,
        pltpu.SemaphoreType.DMA,
        pltpu.SemaphoreType.DMA,
        pltpu.SemaphoreType.DMA,
        pltpu.SemaphoreType.DMA,
        pltpu.SemaphoreType.DMA,
        pltpu.SemaphoreType.DMA,
        pltpu.SemaphoreType.DMA,
        pltpu.SemaphoreType.DMA,
    ],
)
def _edge_kernel(ei_hbm, deg_hbm, xw_hbm,
                 h1p_out, cexp_out,
                 srcf, dstf, didx, tidx, disv, caccv, degstg, ybuf, h1stg,
                 s128buf, rows, tailrows, hout, cout, iota80, h1sh, y_sp,
                 dis_sp, cacc_sp,
                 gs0, gs1, gs2, gs3, ss0, ss1, ss2, ss3,
                 ds0, ds1, ds2, ds3):
    gs = (gs0, gs1, gs2, gs3)
    ss = (ss0, ss1, ss2, ss3)
    dsm = (ds0, ds1, ds2, ds3)
    cid = lax.axis_index("c")
    sid = lax.axis_index("s")
    wid = sid * NC + cid
    ebase = wid * EPT
    nbase = sid * STRIPE

    pltpu.sync_copy(ei_hbm.at[0, pl.ds(ebase, EPT)], srcf)
    pltpu.sync_copy(ei_hbm.at[1, pl.ds(ebase, EPT)], dstf)

    degcps = [
        pltpu.async_copy(
            deg_hbm.at[k, pl.ds(sid * (R128 // NS), R128 // NS)],
            degstg.at[k], gs0)
        for k in range(NC)
    ]
    zero = jnp.full((L,), 0.0, jnp.float32)

    def _zero_cacc(i, _):
        caccv[i // 8, pl.ds((i % 8) * L, L)] = zero
        return 0

    lax.fori_loop(0, NPAD // L, _zero_cacc, 0)

    def _zero_rows(i, _):
        rows[0, i, :] = zero
        return 0

    lax.fori_loop(0, CHUNK, _zero_rows, 0)
    for k in range(STRIPE // CHUNK):
        pltpu.sync_copy(rows.at[0], h1sh.at[pl.ds(nbase + k * CHUNK, CHUNK)])
    for d in degcps:
        d.wait()

    def _dis_stripe(i, _):
        q, o = i // 8, (i % 8) * L
        deg = (degstg[0, q, pl.ds(o, L)] + degstg[1, q, pl.ds(o, L)] + 1.0)
        yh = plsc.bitcast(
            jnp.full((L,), 0x5F3759DF, jnp.int32)
            - (plsc.bitcast(deg, jnp.int32) >> 1),
            jnp.float32)
        half = deg * -0.5
        for _ in range(3):
            yh = yh * (half * yh * yh + 1.5)
        s128buf[i // 8, pl.ds((i % 8) * L, L)] = yh
        return 0

    lax.fori_loop(0, STRIPE // L, _dis_stripe, 0)
    pltpu.sync_copy(s128buf, dis_sp.at[pl.ds(sid * (R128 // NS), R128 // NS)])
    plsc.subcore_barrier()

    pltpu.sync_copy(dis_sp, disv)
    pltpu.sync_copy(xw_hbm.at[pl.ds(nbase, STRIPE)], ybuf)

    def _ymul(g, _):
        nloc = nbase + g * L
        dv = disv[nloc // 128, pl.ds(nloc % 128, L)]
        for t in range(L):
            ybuf[g * L + t, :] = ybuf[g * L + t, :] * dv[t]
        return 0

    lax.fori_loop(0, STRIPE // L, _ymul, 0)
    pltpu.sync_copy(ybuf, y_sp.at[pl.ds(nbase, STRIPE)])
    plsc.subcore_barrier()

    NBUF, LA = 4, 2

    def _start_gather(c, b):
        return pltpu.async_copy(
            y_sp.at[srcf.at[pl.ds(c * CHUNK, CHUNK)]], rows.at[b], gs[b])

    def _start_didx(c, b):
        return pltpu.async_copy(
            ei_hbm.at[1, pl.ds(ebase + c * CHUNK, CHUNK)], didx.at[b], dsm[b])

    def _start_scatter(c, b):
        return pltpu.async_copy(
            rows.at[b], h1sh.at[didx.at[b]], ss[b], add=True)

    def _wait_gather(b):
        pltpu.make_async_copy(
            y_sp.at[srcf.at[pl.ds(0, CHUNK)]], rows.at[b], gs[b]).wait()

    def _wait_didx(b):
        pltpu.make_async_copy(
            ei_hbm.at[1, pl.ds(0, CHUNK)], didx.at[b], dsm[b]).wait()

    def _wait_scatter(b):
        pltpu.make_async_copy(rows.at[b], h1sh.at[didx.at[b]], ss[b]).wait()

    def _cacc_chunk(c):
        @plsc.parallel_loop(0, CHUNK // L, unroll=4)
        def _cacc(g):
            e0 = c * CHUNK + g * L
            di = dstf[pl.ds(e0, L)]
            si = srcf[pl.ds(e0, L)]
            vals = plsc.load_gather(disv, [di >> 7, di & 127])
            plsc.addupdate_scatter(caccv, [si >> 7, si & 127], vals)

    MAIN = (FCPT - LA) // NBUF * NBUF
    for j in range(LA):
        _start_didx(j, j)
        _start_gather(j, j)

    def _main(i, _):
        c0 = i * NBUF
        for j in range(NBUF):
            c = c0 + j
            _wait_gather(j)
            _wait_didx(j)
            _cacc_chunk(c)
            _start_scatter(c, j)
            nb = (j + LA) % NBUF

            @pl.when(c >= LA)
            def _():
                _wait_scatter(nb)

            _start_didx(c + LA, nb)
            _start_gather(c + LA, nb)
        return 0

    lax.fori_loop(0, MAIN // NBUF, _main, 0)

    for c in range(MAIN, FCPT):
        b = c % NBUF
        _wait_gather(b)
        _wait_didx(b)
        _cacc_chunk(c)
        _start_scatter(c, b)
        nc = c + LA
        if nc < FCPT:
            nb = nc % NBUF
            _wait_scatter(nb)
            _start_didx(nc, nb)
            _start_gather(nc, nb)
    pltpu.sync_copy(ei_hbm.at[1, pl.ds(ebase + TAIL_OFF, TAIL)], tidx)
    pltpu.async_copy(
        y_sp.at[srcf.at[pl.ds(TAIL_OFF, TAIL)]], tailrows, gs0).wait()
    pltpu.async_copy(tailrows, h1sh.at[tidx], ss0, add=True).wait()

    @plsc.parallel_loop(0, TAIL // L, unroll=1)
    def _cacc_tail(g):
        e0 = TAIL_OFF + g * L
        di = dstf[pl.ds(e0, L)]
        si = srcf[pl.ds(e0, L)]
        vals = plsc.load_gather(disv, [di >> 7, di & 127])
        plsc.addupdate_scatter(caccv, [si >> 7, si & 127], vals)

    for b in range(NBUF):
        _wait_scatter(b)
    plsc.subcore_barrier()

    @pl.when(wid == 0)
    def _():
        def _sladd(i, _):
            q, o = i // 8, (i % 8) * L
            caccv[q, pl.ds(o, L)] = (caccv[q, pl.ds(o, L)]
                                     + disv[q, pl.ds(o, L)])
            return 0

        lax.fori_loop(0, N // L, _sladd, 0)

    def _cmul(i, _):
        q, o = i // 8, (i % 8) * L
        caccv[q, pl.ds(o, L)] = caccv[q, pl.ds(o, L)] * disv[q, pl.ds(o, L)]
        return 0

    lax.fori_loop(0, NPAD // L, _cmul, 0)
    for k in range(R128 // L):
        iota80[pl.ds(k * L, L)] = lax.iota(jnp.int32, L) + k * L

    @pl.when(sid == 0)
    def _():
        def _zc(i, _):
            s128buf[i // 8, pl.ds((i % 8) * L, L)] = zero
            return 0

        lax.fori_loop(0, STRIPE // L, _zc, 0)
        for k in range(NS):
            pltpu.sync_copy(s128buf,
                            cacc_sp.at[pl.ds(k * (R128 // NS), R128 // NS)])

    plsc.subcore_barrier()
    pltpu.sync_copy(caccv, cacc_sp.at[iota80], add=True)
    plsc.subcore_barrier()

    pltpu.sync_copy(h1sh.at[pl.ds(nbase, STRIPE)], h1stg)
    pltpu.sync_copy(cacc_sp.at[pl.ds(sid * (R128 // NS), R128 // NS)], s128buf)
    yfac = jnp.where(cid == 0, 1.0, 0.0).astype(jnp.float32)

    def _emit(g, _):
        j0 = g * L
        nloc = nbase + j0
        dv = disv[nloc // 128, pl.ds(nloc % 128, L)]
        cv = s128buf[j0 // 128, pl.ds(j0 % 128, L)]
        for t in range(L):
            j = j0 + t
            q = 2 * g + t // 8
            i = t % 8
            h1row = (h1stg[j, :] + ybuf[j, :] * yfac) * dv[t]
            hout[q, pl.ds(i * L, L)] = h1row
            cout[q, pl.ds(i * L, L)] = jnp.zeros((L,), jnp.float32) + cv[t]
        return 0

    lax.fori_loop(0, STRIPE // L, _emit, 0)
    pltpu.sync_copy(hout, h1p_out.at[cid, pl.ds(sid * SROWS, SROWS)])
    pltpu.sync_copy(cout, cexp_out.at[cid, pl.ds(sid * SROWS, SROWS)])


def _fin_body(h1p_ref, cexp_ref, b1_ref, w2_ref, b2_ref, out_ref):
    b1t = jnp.concatenate([b1_ref[...]] * (128 // H), axis=1)
    h1 = jnp.maximum(h1p_ref[0] + h1p_ref[1] + b1t, 0.0)
    ce = cexp_ref[0] + cexp_ref[1]
    s128 = jnp.sum(ce * h1, axis=0, keepdims=True)
    row = lax.broadcasted_iota(jnp.int32, (128, H), 0)
    col = lax.broadcasted_iota(jnp.int32, (128, H), 1)
    p128 = jnp.where(row % H == col, 1.0, 0.0).astype(jnp.float32)
    s16 = jnp.dot(s128, p128,
                  preferred_element_type=jnp.float32)
    logits = jnp.dot(s16 * (1.0 / N), w2_ref[...],
                     preferred_element_type=jnp.float32) + b2_ref[...]
    m = jnp.max(logits)
    ex = jnp.exp(logits - m)
    out_ref[...] = ex / jnp.sum(ex)


def _fin_kernel(h1p, cexp, b1, w2, b2):
    return pl.pallas_call(
        _fin_body,
        out_shape=jax.ShapeDtypeStruct((1, NCLS), jnp.float32),
    )(h1p, cexp, b1, w2, b2)


def kernel(x, edge_index, W1, b1, W2, b2):
    ei = edge_index.astype(jnp.int32)
    deg = _deg_kernel(ei)
    xw = _mm_kernel(x, W1)
    h1p, cexp = _edge_kernel(ei, deg, xw)
    out = _fin_kernel(h1p, cexp, b1.reshape(1, H), W2, b2.reshape(1, NCLS))
    return out.reshape(NCLS)
